# Initial kernel scaffold; baseline (speedup 1.0000x reference)
import functools

import numpy as np
import jax
import jax.numpy as jnp
from jax import lax
from jax.experimental import pallas as pl
from jax.experimental.pallas import tpu as pltpu
from jax.experimental.pallas import tpu_sc as plsc

VOCAB = 100000
EMBED = 128
WINDOW = 2048
BATCH = 4
SEQ = 2048

SCALE = float(np.sqrt(float(EMBED)))

NUM_CORES = 2
NUM_SUBCORES = 16
NW = NUM_CORES * NUM_SUBCORES
TOTAL = BATCH * SEQ
BPW = TOTAL // NW
GCHUNK = 128
NCH = BPW // GCHUNK
LANES = 16


def _pos_encoding() -> np.ndarray:
    depth_h = EMBED / 2
    positions = np.arange(WINDOW)[:, np.newaxis]
    depths = np.arange(depth_h)[np.newaxis, :] / depth_h
    angle_rates = 1 / 10000 ** depths
    angle_rads = positions * angle_rates
    return np.concatenate(
        [np.sin(angle_rads), np.cos(angle_rads)], axis=-1
    ).astype(np.float32)


_PE = jnp.asarray(_pos_encoding())

_MESH = plsc.VectorSubcoreMesh(
    core_axis_name="c", subcore_axis_name="s",
    num_cores=NUM_CORES, num_subcores=NUM_SUBCORES,
)


@functools.partial(
    pl.kernel,
    out_type=jax.ShapeDtypeStruct((TOTAL, EMBED), jnp.float32),
    mesh=_MESH,
    scratch_types=[
        pltpu.VMEM((NCH, GCHUNK), jnp.int32),
        pltpu.VMEM((BPW, EMBED), jnp.float32),
        pltpu.VMEM((BPW, EMBED), jnp.float32),
        pltpu.SemaphoreType.DMA,
    ],
)
def _sc_embed(x_hbm, table_hbm, pe_hbm, out_hbm, idx_v, rows_v, pe_v, sem):
    wid = lax.axis_index("s") * NUM_CORES + lax.axis_index("c")
    base = wid * BPW
    pe_base = lax.rem(base, SEQ)

    pltpu.sync_copy(x_hbm.at[pl.ds(base, BPW)], idx_v)
    gathers = []
    for j in range(NCH):
        gathers.append(
            pltpu.async_copy(
                table_hbm.at[idx_v.at[j]],
                rows_v.at[pl.ds(j * GCHUNK, GCHUNK)],
                sem,
            )
        )
    pltpu.sync_copy(pe_hbm.at[pl.ds(pe_base, BPW)], pe_v)
    for g in gathers:
        g.wait()

    def body(r, carry):
        for j in range(EMBED // LANES):
            sl = pl.ds(j * LANES, LANES)
            rows_v[r, sl] = rows_v[r, sl] * SCALE + pe_v[r, sl]
        return carry

    lax.fori_loop(0, BPW, body, 0)

    pltpu.sync_copy(rows_v, out_hbm.at[pl.ds(base, BPW)])


def kernel(x, table):
    x_flat = x.reshape(TOTAL).astype(jnp.int32)
    out = _sc_embed(x_flat, table, _PE)
    return out.reshape(BATCH, SEQ, EMBED)


# speedup vs baseline: 1.0601x; 1.0601x over previous
import functools

import numpy as np
import jax
import jax.numpy as jnp
from jax import lax
from jax.experimental import pallas as pl
from jax.experimental.pallas import tpu as pltpu
from jax.experimental.pallas import tpu_sc as plsc

VOCAB = 100000
EMBED = 128
WINDOW = 2048
BATCH = 4
SEQ = 2048

SCALE = float(np.sqrt(float(EMBED)))

NUM_CORES = 2
NUM_SUBCORES = 16
NW = NUM_CORES * NUM_SUBCORES
TOTAL = BATCH * SEQ
BPW = TOTAL // NW
GCHUNK = 128
NCH = BPW // GCHUNK
LANES = 16


def _pos_encoding() -> np.ndarray:
    depth_h = EMBED / 2
    positions = np.arange(WINDOW)[:, np.newaxis]
    depths = np.arange(depth_h)[np.newaxis, :] / depth_h
    angle_rates = 1 / 10000 ** depths
    angle_rads = positions * angle_rates
    return np.concatenate(
        [np.sin(angle_rads), np.cos(angle_rads)], axis=-1
    ).astype(np.float32)


_PE_NP = _pos_encoding()

_MESH = plsc.VectorSubcoreMesh(
    core_axis_name="c", subcore_axis_name="s",
    num_cores=NUM_CORES, num_subcores=NUM_SUBCORES,
)


@functools.partial(
    pl.kernel,
    out_type=jax.ShapeDtypeStruct((TOTAL, EMBED), jnp.float32),
    mesh=_MESH,
    scratch_types=[
        pltpu.VMEM((NCH, GCHUNK), jnp.int32),
        pltpu.VMEM((BPW, EMBED), jnp.float32),
        pltpu.VMEM((BPW, EMBED), jnp.float32),
        pltpu.SemaphoreType.DMA,
    ],
)
def _sc_embed(x_hbm, table_hbm, pe_hbm, out_hbm, idx_v, rows_v, pe_v, sem):
    wid = lax.axis_index("s") * NUM_CORES + lax.axis_index("c")
    base = wid * BPW
    pe_base = lax.rem(base, SEQ)

    pltpu.sync_copy(x_hbm.at[wid], idx_v)
    gathers = []
    for j in range(NCH):
        gathers.append(
            pltpu.async_copy(
                table_hbm.at[idx_v.at[j]],
                rows_v.at[pl.ds(j * GCHUNK, GCHUNK)],
                sem,
            )
        )
    pltpu.sync_copy(pe_hbm.at[pl.ds(pe_base, BPW)], pe_v)
    for g in gathers:
        g.wait()

    def body(r, carry):
        for j in range(EMBED // LANES):
            sl = pl.ds(j * LANES, LANES)
            rows_v[r, sl] = rows_v[r, sl] * SCALE + pe_v[r, sl]
        return carry

    lax.fori_loop(0, BPW, body, 0)

    pltpu.sync_copy(rows_v, out_hbm.at[pl.ds(base, BPW)])


def kernel(x, table):
    x_flat = x.reshape(NW, NCH, GCHUNK).astype(jnp.int32)
    out = _sc_embed(x_flat, table, jnp.asarray(_PE_NP))
    return out.reshape(BATCH, SEQ, EMBED)


# speedup vs baseline: 1.1761x; 1.1094x over previous
import functools

import numpy as np
import jax
import jax.numpy as jnp
from jax import lax
from jax.experimental import pallas as pl
from jax.experimental.pallas import tpu as pltpu
from jax.experimental.pallas import tpu_sc as plsc

VOCAB = 100000
EMBED = 128
WINDOW = 2048
BATCH = 4
SEQ = 2048

SCALE = float(np.sqrt(float(EMBED)))

NUM_CORES = 2
NUM_SUBCORES = 16
NW = NUM_CORES * NUM_SUBCORES
PPW = SEQ // NW
LANES = 16
NVEC = EMBED // LANES


def _pos_encoding() -> np.ndarray:
    depth_h = EMBED / 2
    positions = np.arange(WINDOW)[:, np.newaxis]
    depths = np.arange(depth_h)[np.newaxis, :] / depth_h
    angle_rates = 1 / 10000 ** depths
    angle_rads = positions * angle_rates
    return np.concatenate(
        [np.sin(angle_rads), np.cos(angle_rads)], axis=-1
    ).astype(np.float32)


_PE_NP = _pos_encoding()

_MESH = plsc.VectorSubcoreMesh(
    core_axis_name="c", subcore_axis_name="s",
    num_cores=NUM_CORES, num_subcores=NUM_SUBCORES,
)


@functools.partial(
    pl.kernel,
    out_type=jax.ShapeDtypeStruct((BATCH, SEQ, EMBED), jnp.float32),
    mesh=_MESH,
    scratch_types=[
        pltpu.VMEM((BATCH, PPW), jnp.int32),
        pltpu.VMEM((BATCH, PPW, EMBED), jnp.float32),
        pltpu.VMEM((PPW, EMBED), jnp.float32),
        pltpu.SemaphoreType.DMA,
        pltpu.SemaphoreType.DMA,
    ],
)
def _sc_embed(x_hbm, table_hbm, pe_hbm, out_hbm, idx_v, rows_v, pe_v, sem, osem):
    wid = lax.axis_index("s") * NUM_CORES + lax.axis_index("c")
    p0 = wid * PPW

    idx_cps = [
        pltpu.async_copy(x_hbm.at[b, pl.ds(p0, PPW)], idx_v.at[b], sem)
        for b in range(BATCH)
    ]
    for cp in idx_cps:
        cp.wait()

    gathers = [
        pltpu.async_copy(table_hbm.at[idx_v.at[b]], rows_v.at[b], sem)
        for b in range(BATCH)
    ]
    pltpu.sync_copy(pe_hbm.at[pl.ds(p0, PPW)], pe_v)
    for g in gathers:
        g.wait()

    def body(p, carry):
        pe_regs = [pe_v[p, pl.ds(j * LANES, LANES)] for j in range(NVEC)]
        for b in range(BATCH):
            for j in range(NVEC):
                sl = pl.ds(j * LANES, LANES)
                rows_v[b, p, sl] = rows_v[b, p, sl] * SCALE + pe_regs[j]
        return carry

    lax.fori_loop(0, PPW, body, 0)

    outs = [
        pltpu.async_copy(rows_v.at[b], out_hbm.at[b].at[pl.ds(p0, PPW)], osem)
        for b in range(BATCH)
    ]
    for o in outs:
        o.wait()


def kernel(x, table):
    return _sc_embed(x.astype(jnp.int32), table, jnp.asarray(_PE_NP))
